# baseline (speedup 1.0000x reference)
import functools

import jax
import jax.numpy as jnp
from jax import lax
from jax.experimental import pallas as pl
from jax.experimental.pallas import tpu as pltpu
from jax.experimental.pallas import tpu_sc as plsc

N_NODES = 100000
DIM = 128
TYPE_NUM = 300
LANES = 16

NC, NS = 2, 16
NW = NC * NS
CHUNK = 3200
W = 128
NBLK = CHUNK // W
NBUF = 7
NROUND = -(-(NBLK + 1) // NBUF)


def _embed(table, node_type, ls):
    mesh = plsc.VectorSubcoreMesh(core_axis_name="c", subcore_axis_name="s")

    @functools.partial(
        pl.kernel,
        out_type=jax.ShapeDtypeStruct((N_NODES, DIM), jnp.float32),
        mesh=mesh,
        scratch_types=[
            pltpu.VMEM_SHARED((TYPE_NUM, DIM), jnp.float32),
            pltpu.VMEM((CHUNK,), jnp.int32),
            pltpu.VMEM((CHUNK,), jnp.int32),
            pltpu.VMEM((CHUNK,), jnp.int32),
            pltpu.VMEM((1, DIM), jnp.float32),
            [pltpu.VMEM((W, DIM), jnp.float32) for _ in range(NBUF)],
            [pltpu.SemaphoreType.DMA for _ in range(NBUF)],
            [pltpu.SemaphoreType.DMA for _ in range(NBUF)],
        ],
    )
    def k(t_hbm, nt_hbm, ls_hbm, out_hbm, tab_v, nt_v, ls_v, idx_v, zrow,
          rows, wsem, gsem):
        wid = lax.axis_index("s") * NC + lax.axis_index("c")

        @pl.when(lax.axis_index("s") == 0)
        def _():
            pltpu.sync_copy(t_hbm, tab_v)

            @pl.loop(0, DIM, step=LANES)
            def _(j):
                zrow[0, pl.ds(j, LANES)] = jnp.zeros((LANES,), jnp.float32)

            pltpu.sync_copy(zrow, tab_v.at[pl.ds(0, 1)])

        base = jnp.minimum(wid * CHUNK, N_NODES - CHUNK)
        pltpu.sync_copy(nt_hbm.at[pl.ds(base, CHUNK)], nt_v)
        pltpu.sync_copy(ls_hbm.at[pl.ds(base, CHUNK)], ls_v)

        @pl.loop(0, CHUNK, step=LANES)
        def _(j):
            sl = pl.ds(j, LANES)
            idx_v[sl] = nt_v[sl] + ls_v[sl] * 100

        plsc.subcore_barrier()

        def fire_gather(i, b):
            pltpu.async_copy(
                tab_v.at[idx_v.at[pl.ds(i * W, W)]], rows[b], gsem[b]
            )

        def wait_gather(i, b):
            pltpu.make_async_copy(
                tab_v.at[idx_v.at[pl.ds(i * W, W)]], rows[b], gsem[b]
            ).wait()

        @pl.loop(0, NROUND)
        def _(i0):
            for b in range(NBUF):
                i = i0 * NBUF + b
                bp = (b - 3) % NBUF

                @pl.when(i < NBLK)
                def _():
                    @pl.when(i0 > 0)
                    def _():
                        pltpu.make_async_copy(
                            rows[b], out_hbm.at[pl.ds(0, W)], wsem[b]
                        ).wait()

                    fire_gather(i, b)

                @pl.when((i >= 3) & (i <= NBLK + 2))
                def _():
                    wait_gather(i - 3, bp)
                    pltpu.async_copy(
                        rows[bp], out_hbm.at[pl.ds(base + (i - 3) * W, W)],
                        wsem[bp],
                    )

        for b in range(NBUF):
            pltpu.make_async_copy(
                rows[b], out_hbm.at[pl.ds(0, W)], wsem[b]
            ).wait()

    return k(table, node_type, ls)


def kernel(node_type, ls, table):
    return _embed(table, node_type.astype(jnp.int32), ls.astype(jnp.int32))


# speedup vs baseline: 1.0097x; 1.0097x over previous
import functools

import jax
import jax.numpy as jnp
from jax import lax
from jax.experimental import pallas as pl
from jax.experimental.pallas import tpu as pltpu
from jax.experimental.pallas import tpu_sc as plsc

N_NODES = 100000
DIM = 128
TYPE_NUM = 300
LANES = 16

NC, NS = 2, 16
NW = NC * NS
CHUNK = 3200
W = 64
NBLK = CHUNK // W
NBUF = 10
NROUND = -(-(NBLK + 2) // NBUF)


def _embed(table, node_type, ls):
    mesh = plsc.VectorSubcoreMesh(core_axis_name="c", subcore_axis_name="s")

    @functools.partial(
        pl.kernel,
        out_type=jax.ShapeDtypeStruct((N_NODES, DIM), jnp.float32),
        mesh=mesh,
        scratch_types=[
            pltpu.VMEM_SHARED((TYPE_NUM, DIM), jnp.float32),
            pltpu.VMEM((CHUNK,), jnp.int32),
            pltpu.VMEM((CHUNK,), jnp.int32),
            pltpu.VMEM((CHUNK,), jnp.int32),
            pltpu.VMEM((1, DIM), jnp.float32),
            [pltpu.VMEM((W, DIM), jnp.float32) for _ in range(NBUF)],
            [pltpu.SemaphoreType.DMA for _ in range(NBUF)],
            [pltpu.SemaphoreType.DMA for _ in range(NBUF)],
        ],
    )
    def k(t_hbm, nt_hbm, ls_hbm, out_hbm, tab_v, nt_v, ls_v, idx_v, zrow,
          rows, wsem, gsem):
        wid = lax.axis_index("s") * NC + lax.axis_index("c")

        @pl.when(lax.axis_index("s") == 0)
        def _():
            pltpu.sync_copy(t_hbm, tab_v)

            @pl.loop(0, DIM, step=LANES)
            def _(j):
                zrow[0, pl.ds(j, LANES)] = jnp.zeros((LANES,), jnp.float32)

            pltpu.sync_copy(zrow, tab_v.at[pl.ds(0, 1)])

        base = jnp.minimum(wid * CHUNK, N_NODES - CHUNK)
        pltpu.sync_copy(nt_hbm.at[pl.ds(base, CHUNK)], nt_v)
        pltpu.sync_copy(ls_hbm.at[pl.ds(base, CHUNK)], ls_v)

        @pl.loop(0, CHUNK, step=LANES)
        def _(j):
            sl = pl.ds(j, LANES)
            idx_v[sl] = nt_v[sl] + ls_v[sl] * 100

        plsc.subcore_barrier()

        def fire_gather(i, b):
            pltpu.async_copy(
                tab_v.at[idx_v.at[pl.ds(i * W, W)]], rows[b], gsem[b]
            )

        def wait_gather(i, b):
            pltpu.make_async_copy(
                tab_v.at[idx_v.at[pl.ds(i * W, W)]], rows[b], gsem[b]
            ).wait()

        @pl.loop(0, NROUND)
        def _(i0):
            for b in range(NBUF):
                i = i0 * NBUF + b
                bp = (b - 2) % NBUF

                @pl.when(i < NBLK)
                def _():
                    @pl.when(i0 > 0)
                    def _():
                        pltpu.make_async_copy(
                            rows[b], out_hbm.at[pl.ds(0, W)], wsem[b]
                        ).wait()

                    fire_gather(i, b)

                @pl.when((i >= 2) & (i <= NBLK + 1))
                def _():
                    wait_gather(i - 2, bp)
                    pltpu.async_copy(
                        rows[bp], out_hbm.at[pl.ds(base + (i - 2) * W, W)],
                        wsem[bp],
                    )

        for b in range(NBUF):
            pltpu.make_async_copy(
                rows[b], out_hbm.at[pl.ds(0, W)], wsem[b]
            ).wait()

    return k(table, node_type, ls)


def kernel(node_type, ls, table):
    return _embed(table, node_type.astype(jnp.int32), ls.astype(jnp.int32))
